# baseline (speedup 1.0000x reference)
import functools

import jax
import jax.numpy as jnp
from jax import lax
from jax.experimental import pallas as pl
from jax.experimental.pallas import tpu as pltpu
from jax.experimental.pallas import tpu_sc as plsc

NC = 2
NS = 16
NW = NC * NS
LANES = 16
NBUF = 4


@functools.lru_cache(maxsize=None)
def _build(B, L, V, D):
    per_w = B // NW
    CB = 4
    n_chunks = per_w // CB
    C = CB * L
    assert per_w % CB == 0 and B % NW == 0 and D == 2 * LANES
    assert n_chunks % NBUF == 0 and n_chunks >= 2 * NBUF

    mesh = plsc.VectorSubcoreMesh(core_axis_name="c", subcore_axis_name="s")

    @functools.partial(
        pl.kernel,
        mesh=mesh,
        compiler_params=pltpu.CompilerParams(use_tc_tiling_on_sc=False),
        out_type=jax.ShapeDtypeStruct((B, L, D), jnp.float32),
        scratch_types=(
            [pltpu.VMEM((CB, L), jnp.int32) for _ in range(NBUF)]
            + [pltpu.VMEM((CB, L, D), jnp.float32) for _ in range(NBUF)]
            + [pltpu.VMEM((L, D), jnp.float32)]
            + [pltpu.SemaphoreType.DMA for _ in range(2 * NBUF)]
        ),
    )
    def k(tok_hbm, idx_hbm, pos_hbm, out_hbm, *refs):
        idx_v = refs[0:NBUF]
        rows_v = refs[NBUF:2 * NBUF]
        pos_v = refs[2 * NBUF]
        gsem = refs[2 * NBUF + 1:2 * NBUF + 1 + NBUF]
        ssem = refs[2 * NBUF + 1 + NBUF:2 * NBUF + 1 + 2 * NBUF]

        wid = lax.axis_index("s") * NC + lax.axis_index("c")
        base = wid * per_w
        pltpu.sync_copy(pos_hbm, pos_v)

        def start_gathers(ci, b):
            boff = base + ci * CB
            pltpu.sync_copy(idx_hbm.at[pl.ds(boff, CB)], idx_v[b])
            for cb in range(CB):
                pltpu.make_async_copy(
                    tok_hbm.at[idx_v[b].at[cb]], rows_v[b].at[cb],
                    gsem[b]).start()

        def wait_gathers(b):
            for cb in range(CB):
                pltpu.make_async_copy(
                    tok_hbm.at[idx_v[b].at[cb]], rows_v[b].at[cb],
                    gsem[b]).wait()

        start_gathers(0, 0)
        start_gathers(1, 1)

        def quad_body(pi, _):
            for b in range(NBUF):
                ci = NBUF * pi + b
                boff = base + ci * CB
                wait_gathers(b)

                def add_l(li, _):
                    p0 = pos_v[li, pl.ds(0, LANES)]
                    p1 = pos_v[li, pl.ds(LANES, LANES)]
                    for cb in range(CB):
                        rows_v[b][cb, li, pl.ds(0, LANES)] = (
                            rows_v[b][cb, li, pl.ds(0, LANES)] + p0)
                        rows_v[b][cb, li, pl.ds(LANES, LANES)] = (
                            rows_v[b][cb, li, pl.ds(LANES, LANES)] + p1)
                    return 0

                lax.fori_loop(0, L, add_l, 0, unroll=4)
                pltpu.make_async_copy(
                    rows_v[b], out_hbm.at[pl.ds(boff, CB)], ssem[b]).start()

                b2 = (b + 2) % NBUF

                @pl.when(ci >= 2)
                def _():
                    boff_prev = base + (ci - 2) * CB
                    pltpu.make_async_copy(
                        rows_v[b2], out_hbm.at[pl.ds(boff_prev, CB)],
                        ssem[b2]).wait()

                @pl.when(ci + 2 < n_chunks)
                def _():
                    start_gathers(ci + 2, b2)

            return 0

        lax.fori_loop(0, n_chunks // NBUF, quad_body, 0)

        for ci in (n_chunks - 2, n_chunks - 1):
            b = ci % NBUF
            boff = base + ci * CB
            pltpu.make_async_copy(
                rows_v[b], out_hbm.at[pl.ds(boff, CB)], ssem[b]).wait()

    return k


def kernel(x, token_table, pos_table):
    B, L = x.shape
    V, D = token_table.shape
    k = _build(B, L, V, D)
    return k(token_table, x, pos_table)
